# baseline (speedup 1.0000x reference)
import functools

import jax
import jax.numpy as jnp
from jax import lax
from jax.experimental import pallas as pl
from jax.experimental.pallas import tpu as pltpu
from jax.experimental.pallas import tpu_sc as plsc

V = 1000000
B = 16384
D = 64
DP = 128
NC = 2
NS = 16
NW = NC * NS
BPW = B // NW
CH = 128
NCH = BPW // CH

IB = 8192
IBL = 13
NBLK = (V + IB - 1) // IB
NPAIR = (NBLK + 1) // 2
VP = NPAIR * IB


def _detile_body(in_ref, out_ref):
    x = in_ref[...]
    xc = jnp.concatenate([x[:, :IB], x[:, IB:]], axis=0)
    eye = (lax.broadcasted_iota(jnp.int32, (DP, DP), 0)
           == lax.broadcasted_iota(jnp.int32, (DP, DP), 1)
           ).astype(jnp.float32)
    out_ref[...] = lax.dot_general(
        xc, eye, (((0,), (0,)), ((), ())),
        precision=lax.Precision.HIGHEST,
        preferred_element_type=jnp.float32)


def _detile(emb_t):
    return pl.pallas_call(
        _detile_body,
        grid=(NPAIR,),
        compiler_params=pltpu.CompilerParams(
            fuse_transposed_lhs_in_matmul=True),
        in_specs=[pl.BlockSpec((D, 2 * IB), lambda i: (0, i))],
        out_specs=pl.BlockSpec((IB, DP), lambda i: (i, 0)),
        out_shape=jax.ShapeDtypeStruct((VP, DP), jnp.float32),
    )(emb_t)


def _compute_chunk(po_s, po_t, rows_s, rows_t, out_v, j):
    lane = lax.iota(jnp.int32, 16)
    for g in range(CH // 16):
        res = jnp.zeros((16,), jnp.float32)
        ps = po_s[j, pl.ds(g * 16, 16)]
        pt = po_t[j, pl.ds(g * 16, 16)]
        for b in range(16):
            p = g * 16 + b
            off_s = pl.multiple_of(ps[b], 64)
            off_t = pl.multiple_of(pt[b], 64)
            acc = jnp.zeros((16,), jnp.float32)
            for c in range(D // 16):
                s = rows_s[p, pl.ds(off_s + c * 16, 16)]
                t = rows_t[p, pl.ds(off_t + c * 16, 16)]
                acc = acc + s * t
            res = jnp.where(lane == b, jnp.sum(acc), res)
        out_v[pl.ds(j * CH + g * 16, 16)] = res


def _sc_body(srch_hbm, trgh_hbm, spar_hbm, tpar_hbm, emb_hbm, out_hbm,
             idx_s, idx_t, po_s, po_t, bS0, bT0, bS1, bT1, out_v, sem):
    wid = lax.axis_index("s") * NC + lax.axis_index("c")
    base_row = wid * NCH

    pltpu.sync_copy(srch_hbm.at[pl.ds(base_row, NCH)], idx_s)
    pltpu.sync_copy(trgh_hbm.at[pl.ds(base_row, NCH)], idx_t)
    pltpu.sync_copy(spar_hbm.at[pl.ds(base_row, NCH)], po_s)
    pltpu.sync_copy(tpar_hbm.at[pl.ds(base_row, NCH)], po_t)

    def fire(j, bufS, bufT):
        pltpu.async_copy(emb_hbm.at[idx_s.at[j]], bufS, sem)
        pltpu.async_copy(emb_hbm.at[idx_t.at[j]], bufT, sem)

    def drain(bufS, bufT):
        pltpu.make_async_copy(emb_hbm.at[idx_s.at[0]], bufS, sem).wait()
        pltpu.make_async_copy(emb_hbm.at[idx_t.at[0]], bufT, sem).wait()

    fire(0, bS0, bT0)
    fire(1, bS1, bT1)
    drain(bS0, bT0)
    _compute_chunk(po_s, po_t, bS0, bT0, out_v, 0)
    fire(2, bS0, bT0)
    drain(bS1, bT1)
    _compute_chunk(po_s, po_t, bS1, bT1, out_v, 1)
    fire(3, bS1, bT1)
    drain(bS0, bT0)
    _compute_chunk(po_s, po_t, bS0, bT0, out_v, 2)
    drain(bS1, bT1)
    _compute_chunk(po_s, po_t, bS1, bT1, out_v, 3)

    pltpu.sync_copy(out_v, out_hbm.at[pl.ds(wid * BPW, BPW)])


_sc_kernel = functools.partial(
    pl.kernel,
    out_type=jax.ShapeDtypeStruct((B,), jnp.float32),
    mesh=plsc.VectorSubcoreMesh(core_axis_name="c", subcore_axis_name="s"),
    compiler_params=pltpu.CompilerParams(
        needs_layout_passes=False, use_tc_tiling_on_sc=True),
    scratch_types=[
        pltpu.VMEM((NCH, CH), jnp.int32),
        pltpu.VMEM((NCH, CH), jnp.int32),
        pltpu.VMEM((NCH, CH), jnp.int32),
        pltpu.VMEM((NCH, CH), jnp.int32),
        pltpu.VMEM((CH, DP), jnp.float32),
        pltpu.VMEM((CH, DP), jnp.float32),
        pltpu.VMEM((CH, DP), jnp.float32),
        pltpu.VMEM((CH, DP), jnp.float32),
        pltpu.VMEM((BPW,), jnp.float32),
        pltpu.SemaphoreType.DMA,
    ],
)(_sc_body)


def kernel(network, src, trg, emb):
    full_cols = (NBLK - 1) * IB

    def packed_row(x):
        blk = x >> IBL
        j_main = ((blk >> 1) << IBL) | (x & (IB - 1))
        j_tail = (NPAIR - 1) * IB + (x - full_cols)
        return jnp.where(x < full_cols, j_main, j_tail)

    def lane_off(x):
        tail_off = ((NBLK - 1) & 1) << 6
        return jnp.where(x < full_cols, ((x >> IBL) & 1) << 6, tail_off)

    src32 = src.astype(jnp.int32)
    trg32 = trg.astype(jnp.int32)
    srch = packed_row(src32).reshape(NW * NCH, CH)
    trgh = packed_row(trg32).reshape(NW * NCH, CH)
    spar = lane_off(src32).reshape(NW * NCH, CH)
    tpar = lane_off(trg32).reshape(NW * NCH, CH)
    packed = _detile(emb.T)
    return _sc_kernel(srch, trgh, spar, tpar, packed)


# speedup vs baseline: 1.3589x; 1.3589x over previous
import functools

import jax
import jax.numpy as jnp
from jax import lax
from jax.experimental import pallas as pl
from jax.experimental.pallas import tpu as pltpu
from jax.experimental.pallas import tpu_sc as plsc

V = 1000000
B = 16384
D = 64
DP = 128
NC = 2
NS = 16
NW = NC * NS
BPW = B // NW
CH = 128
NCH = BPW // CH

IB = 16384
IBL = 14
NBLK = (V + IB - 1) // IB
NPAIR = (NBLK + 1) // 2
VP = NPAIR * IB


def _detile_body(in_ref, out_ref):
    x = in_ref[...]
    xc = jnp.concatenate([x[:, :IB], x[:, IB:]], axis=0)
    eye = (lax.broadcasted_iota(jnp.int32, (DP, DP), 0)
           == lax.broadcasted_iota(jnp.int32, (DP, DP), 1)
           ).astype(jnp.float32)
    out_ref[...] = lax.dot_general(
        xc, eye, (((0,), (0,)), ((), ())),
        preferred_element_type=jnp.float32)


def _detile(emb_t):
    return pl.pallas_call(
        _detile_body,
        grid=(NPAIR,),
        compiler_params=pltpu.CompilerParams(
            fuse_transposed_lhs_in_matmul=True,
            vmem_limit_bytes=100 * 1024 * 1024),
        in_specs=[pl.BlockSpec((D, 2 * IB), lambda i: (0, i))],
        out_specs=pl.BlockSpec((IB, DP), lambda i: (i, 0)),
        out_shape=jax.ShapeDtypeStruct((VP, DP), jnp.float32),
    )(emb_t)


def _compute_chunk(po_s, po_t, rows_s, rows_t, out_v, j):
    lane = lax.iota(jnp.int32, 16)
    for g in range(CH // 16):
        res = jnp.zeros((16,), jnp.float32)
        ps = po_s[j, pl.ds(g * 16, 16)]
        pt = po_t[j, pl.ds(g * 16, 16)]
        for b in range(16):
            p = g * 16 + b
            off_s = pl.multiple_of(ps[b], 64)
            off_t = pl.multiple_of(pt[b], 64)
            acc = jnp.zeros((16,), jnp.float32)
            for c in range(D // 16):
                s = rows_s[p, pl.ds(off_s + c * 16, 16)]
                t = rows_t[p, pl.ds(off_t + c * 16, 16)]
                acc = acc + s * t
            res = jnp.where(lane == b, jnp.sum(acc), res)
        out_v[pl.ds(j * CH + g * 16, 16)] = res


def _sc_body(srch_hbm, trgh_hbm, spar_hbm, tpar_hbm, emb_hbm, out_hbm,
             idx_s, idx_t, po_s, po_t, bS0, bT0, bS1, bT1, out_v, sem):
    wid = lax.axis_index("s") * NC + lax.axis_index("c")
    base_row = wid * NCH

    pltpu.sync_copy(srch_hbm.at[pl.ds(base_row, NCH)], idx_s)
    pltpu.sync_copy(trgh_hbm.at[pl.ds(base_row, NCH)], idx_t)
    pltpu.sync_copy(spar_hbm.at[pl.ds(base_row, NCH)], po_s)
    pltpu.sync_copy(tpar_hbm.at[pl.ds(base_row, NCH)], po_t)

    def fire(j, bufS, bufT):
        pltpu.async_copy(emb_hbm.at[idx_s.at[j]], bufS, sem)
        pltpu.async_copy(emb_hbm.at[idx_t.at[j]], bufT, sem)

    def drain(bufS, bufT):
        pltpu.make_async_copy(emb_hbm.at[idx_s.at[0]], bufS, sem).wait()
        pltpu.make_async_copy(emb_hbm.at[idx_t.at[0]], bufT, sem).wait()

    fire(0, bS0, bT0)
    fire(1, bS1, bT1)
    drain(bS0, bT0)
    _compute_chunk(po_s, po_t, bS0, bT0, out_v, 0)
    fire(2, bS0, bT0)
    drain(bS1, bT1)
    _compute_chunk(po_s, po_t, bS1, bT1, out_v, 1)
    fire(3, bS1, bT1)
    drain(bS0, bT0)
    _compute_chunk(po_s, po_t, bS0, bT0, out_v, 2)
    drain(bS1, bT1)
    _compute_chunk(po_s, po_t, bS1, bT1, out_v, 3)

    pltpu.sync_copy(out_v, out_hbm.at[pl.ds(wid * BPW, BPW)])


_sc_kernel = functools.partial(
    pl.kernel,
    out_type=jax.ShapeDtypeStruct((B,), jnp.float32),
    mesh=plsc.VectorSubcoreMesh(core_axis_name="c", subcore_axis_name="s"),
    compiler_params=pltpu.CompilerParams(
        needs_layout_passes=False, use_tc_tiling_on_sc=True),
    scratch_types=[
        pltpu.VMEM((NCH, CH), jnp.int32),
        pltpu.VMEM((NCH, CH), jnp.int32),
        pltpu.VMEM((NCH, CH), jnp.int32),
        pltpu.VMEM((NCH, CH), jnp.int32),
        pltpu.VMEM((CH, DP), jnp.float32),
        pltpu.VMEM((CH, DP), jnp.float32),
        pltpu.VMEM((CH, DP), jnp.float32),
        pltpu.VMEM((CH, DP), jnp.float32),
        pltpu.VMEM((BPW,), jnp.float32),
        pltpu.SemaphoreType.DMA,
    ],
)(_sc_body)


def kernel(network, src, trg, emb):
    full_cols = (NBLK - 1) * IB

    def packed_row(x):
        blk = x >> IBL
        j_main = ((blk >> 1) << IBL) | (x & (IB - 1))
        j_tail = (NPAIR - 1) * IB + (x - full_cols)
        return jnp.where(x < full_cols, j_main, j_tail)

    def lane_off(x):
        tail_off = ((NBLK - 1) & 1) << 6
        return jnp.where(x < full_cols, ((x >> IBL) & 1) << 6, tail_off)

    src32 = src.astype(jnp.int32)
    trg32 = trg.astype(jnp.int32)
    srch = packed_row(src32).reshape(NW * NCH, CH)
    trgh = packed_row(trg32).reshape(NW * NCH, CH)
    spar = lane_off(src32).reshape(NW * NCH, CH)
    tpar = lane_off(trg32).reshape(NW * NCH, CH)
    packed = _detile(emb.T)
    return _sc_kernel(srch, trgh, spar, tpar, packed)
